# Initial kernel scaffold; baseline (speedup 1.0000x reference)
import jax
import jax.numpy as jnp
from jax.experimental import pallas as pl


def kernel(h, edge_index, edge_len, W1, b1, W2, b2):
    raise NotImplementedError("write your pallas kernel here")


# speedup vs baseline: 2.6381x; 2.6381x over previous
import functools

import jax
import jax.numpy as jnp
from jax import lax
from jax.experimental import pallas as pl
from jax.experimental.pallas import tpu as pltpu
from jax.experimental.pallas import tpu_sc as plsc

HIDDEN = 128
N_NODES = 10000
E_PAD = 327680
G = 2 * E_PAD
ACC_ROWS = 10240
EB = 2048
NB = E_PAD // EB
W = 128
N_SUBCORES = 16

def _sc_mesh():
    return plsc.VectorSubcoreMesh(core_axis_name="c", subcore_axis_name="s")


def _gather(table, idx):
    @functools.partial(
        pl.kernel,
        out_type=jax.ShapeDtypeStruct((G, HIDDEN), jnp.float32),
        mesh=_sc_mesh(),
    )
    def kern(table_hbm, idx_hbm, out_hbm):
        def body(i_vmem, o_vmem):
            pltpu.sync_copy(table_hbm.at[i_vmem.at[0]], o_vmem)

        pltpu.emit_pipeline(
            body,
            grid=(G // W,),
            in_specs=[pl.BlockSpec((1, W), lambda i: (0, i))],
            out_specs=[pl.BlockSpec((W, HIDDEN), lambda i: (i, 0))],
            core_axis_name=("c", "s"),
            dimension_semantics=(pltpu.PARALLEL,),
        )(idx_hbm, out_hbm)

    return kern(table, idx)


def _scatter_add(msgs, ridx, zeros):
    @functools.partial(
        pl.kernel,
        out_type=jax.ShapeDtypeStruct((2, ACC_ROWS, HIDDEN), jnp.float32),
        mesh=_sc_mesh(),
        scratch_types=[pltpu.VMEM_SHARED((ACC_ROWS, HIDDEN), jnp.float32)],
    )
    def kern(msgs_hbm, ridx_hbm, zeros_hbm, part_hbm, acc):
        c = lax.axis_index("c")
        s = lax.axis_index("s")
        stripe = ACC_ROWS // N_SUBCORES
        r0 = s * stripe
        pltpu.sync_copy(zeros_hbm.at[pl.ds(r0, stripe)], acc.at[pl.ds(r0, stripe)])
        plsc.subcore_barrier()

        def body(m_vmem, i_vmem):
            pltpu.sync_copy(m_vmem, acc.at[i_vmem.at[0]], add=True)

        pltpu.emit_pipeline(
            body,
            grid=(E_PAD // W,),
            in_specs=[
                pl.BlockSpec((W, HIDDEN), lambda i: (i, 0)),
                pl.BlockSpec((1, W), lambda i: (0, i)),
            ],
            out_specs=[],
            core_axis_name=("c", "s"),
            dimension_semantics=(pltpu.PARALLEL,),
        )(msgs_hbm, ridx_hbm)

        plsc.subcore_barrier()
        pltpu.sync_copy(acc.at[pl.ds(r0, stripe)], part_hbm.at[c, pl.ds(r0, stripe)])

    return kern(msgs, ridx, zeros)


def _mlp_body(gs_ref, gr_ref, el_ref, w1s_ref, w1r_ref, w1e_ref, b1_ref,
              w2_ref, b2_ref, o_ref):
    dn_t = (((0,), (1,)), ((), ()))
    x = lax.dot_general(w1s_ref[...], gs_ref[...], dn_t,
                        preferred_element_type=jnp.float32)
    x += lax.dot_general(w1r_ref[...], gr_ref[...], dn_t,
                         preferred_element_type=jnp.float32)
    el = el_ref[0]
    x += lax.dot_general(w1e_ref[...], el, (((0,), (0,)), ((), ())),
                         preferred_element_type=jnp.float32)
    x += b1_ref[...]
    hmid = x * jax.nn.sigmoid(x)
    m = lax.dot_general(hmid, w2_ref[...], (((0,), (0,)), ((), ())),
                        preferred_element_type=jnp.float32)
    o_ref[...] = m + b2_ref[...]


def _mlp(gathered, el3, w1s, w1r, w1e, b1c, w2, b2r):
    return pl.pallas_call(
        _mlp_body,
        grid=(NB,),
        in_specs=[
            pl.BlockSpec((EB, HIDDEN), lambda i: (i, 0)),
            pl.BlockSpec((EB, HIDDEN), lambda i: (i + NB, 0)),
            pl.BlockSpec((1, 1, EB), lambda i: (i, 0, 0)),
            pl.BlockSpec((HIDDEN, HIDDEN), lambda i: (0, 0)),
            pl.BlockSpec((HIDDEN, HIDDEN), lambda i: (0, 0)),
            pl.BlockSpec((1, HIDDEN), lambda i: (0, 0)),
            pl.BlockSpec((HIDDEN, 1), lambda i: (0, 0)),
            pl.BlockSpec((HIDDEN, HIDDEN), lambda i: (0, 0)),
            pl.BlockSpec((1, HIDDEN), lambda i: (0, 0)),
        ],
        out_specs=pl.BlockSpec((EB, HIDDEN), lambda i: (i, 0)),
        out_shape=jax.ShapeDtypeStruct((E_PAD, HIDDEN), jnp.float32),
    )(gathered, gathered, el3, w1s, w1r, w1e, b1c, w2, b2r)


def _final_body(h_ref, p_ref, o_ref):
    o_ref[:, :HIDDEN] = h_ref[:, :HIDDEN] + p_ref[0] + p_ref[1]
    o_ref[:, HIDDEN:] = h_ref[:, HIDDEN:]


def _finalize(h, part):
    n, f = h.shape
    rb = 1000
    return pl.pallas_call(
        _final_body,
        grid=(n // rb,),
        in_specs=[
            pl.BlockSpec((rb, f), lambda i: (i, 0)),
            pl.BlockSpec((2, rb, HIDDEN), lambda i: (0, i, 0)),
        ],
        out_specs=pl.BlockSpec((rb, f), lambda i: (i, 0)),
        out_shape=jax.ShapeDtypeStruct((n, f), jnp.float32),
    )(h, part)


def kernel(h, edge_index, edge_len, W1, b1, W2, b2):
    scalars = h[:, :HIDDEN]
    sender = edge_index[0].astype(jnp.int32)
    receiver = edge_index[1].astype(jnp.int32)
    e = sender.shape[0]
    pad = E_PAD - e
    sender_p = jnp.pad(sender, (0, pad))
    receiver_p = jnp.pad(receiver, (0, pad), constant_values=N_NODES)
    el_p = jnp.pad(edge_len.astype(jnp.float32), (0, pad))
    idx_all = jnp.concatenate([sender_p, receiver_p]).reshape(1, G)

    gathered = _gather(scalars, idx_all)
    msgs = _mlp(
        gathered,
        el_p.reshape(NB, 1, EB),
        W1[:HIDDEN],
        W1[HIDDEN:2 * HIDDEN],
        W1[2 * HIDDEN:],
        b1.reshape(HIDDEN, 1),
        W2,
        b2.reshape(1, HIDDEN),
    )
    part = _scatter_add(
        msgs,
        receiver_p.reshape(1, E_PAD),
        jnp.zeros((ACC_ROWS, HIDDEN), jnp.float32),
    )
    return _finalize(h, part)


# speedup vs baseline: 2.8993x; 1.0990x over previous
import functools

import jax
import jax.numpy as jnp
from jax import lax
from jax.experimental import pallas as pl
from jax.experimental.pallas import tpu as pltpu
from jax.experimental.pallas import tpu_sc as plsc

HIDDEN = 128
N_NODES = 10000
E_PAD = 327680
G = 2 * E_PAD
ACC_ROWS = 10240
EB = 2048
NB = E_PAD // EB
W = 128
N_SUBCORES = 16
NW = 2 * N_SUBCORES
WSTEPS = G // W // NW
NBUF = 4


def _sc_mesh():
    return plsc.VectorSubcoreMesh(core_axis_name="c", subcore_axis_name="s")


def _gather(table, idx):
    @functools.partial(
        pl.kernel,
        out_type=jax.ShapeDtypeStruct((G, HIDDEN), jnp.float32),
        mesh=_sc_mesh(),
        scratch_types=[
            pltpu.VMEM((WSTEPS, W), jnp.int32),
            pltpu.VMEM((NBUF, W, HIDDEN), jnp.float32),
            pltpu.SemaphoreType.DMA((NBUF,)),
            pltpu.SemaphoreType.DMA((NBUF,)),
        ],
    )
    def kern(table_hbm, idx_hbm, out_hbm, idx_v, bufs, gsem, osem):
        wid = lax.axis_index("c") * N_SUBCORES + lax.axis_index("s")
        pltpu.sync_copy(idx_hbm.at[pl.ds(wid * WSTEPS, WSTEPS)], idx_v)

        def out_slot(w):
            return out_hbm.at[pl.ds((wid * WSTEPS + w) * W, W)]

        for b in range(NBUF):
            pltpu.async_copy(table_hbm.at[idx_v.at[b]], bufs.at[b], gsem.at[b])

        @pl.loop(0, WSTEPS // NBUF)
        def _(k):
            for b in range(NBUF):
                w = k * NBUF + b
                pltpu.make_async_copy(
                    table_hbm.at[idx_v.at[w]], bufs.at[b], gsem.at[b]).wait()
                pltpu.async_copy(bufs.at[b], out_slot(w), osem.at[b])

                @pl.when(k < WSTEPS // NBUF - 1)
                def _():
                    pltpu.make_async_copy(
                        bufs.at[b], out_slot(w), osem.at[b]).wait()
                    pltpu.async_copy(table_hbm.at[idx_v.at[w + NBUF]],
                                     bufs.at[b], gsem.at[b])

        for b in range(NBUF):
            pltpu.make_async_copy(
                bufs.at[b], out_slot(WSTEPS - NBUF + b), osem.at[b]).wait()

    return kern(table, idx)


def _scatter_add(msgs, ridx, zeros):
    @functools.partial(
        pl.kernel,
        out_type=jax.ShapeDtypeStruct((2, ACC_ROWS, HIDDEN), jnp.float32),
        mesh=_sc_mesh(),
        scratch_types=[pltpu.VMEM_SHARED((ACC_ROWS, HIDDEN), jnp.float32)],
    )
    def kern(msgs_hbm, ridx_hbm, zeros_hbm, part_hbm, acc):
        c = lax.axis_index("c")
        s = lax.axis_index("s")
        stripe = ACC_ROWS // N_SUBCORES
        r0 = s * stripe
        pltpu.sync_copy(zeros_hbm.at[pl.ds(r0, stripe)], acc.at[pl.ds(r0, stripe)])
        plsc.subcore_barrier()

        def body(m_vmem, i_vmem):
            pltpu.sync_copy(m_vmem, acc.at[i_vmem.at[0]], add=True)

        pltpu.emit_pipeline(
            body,
            grid=(E_PAD // W,),
            in_specs=[
                pl.BlockSpec((W, HIDDEN), lambda i: (i, 0)),
                pl.BlockSpec((1, W), lambda i: (0, i)),
            ],
            out_specs=[],
            core_axis_name=("c", "s"),
            dimension_semantics=(pltpu.PARALLEL,),
        )(msgs_hbm, ridx_hbm)

        plsc.subcore_barrier()
        pltpu.sync_copy(acc.at[pl.ds(r0, stripe)], part_hbm.at[c, pl.ds(r0, stripe)])

    return kern(msgs, ridx, zeros)


def _mlp_body(gs_ref, gr_ref, el_ref, w1s_ref, w1r_ref, w1e_ref, b1_ref,
              w2_ref, b2_ref, o_ref):
    dn_t = (((0,), (1,)), ((), ()))
    x = lax.dot_general(w1s_ref[...], gs_ref[...].astype(jnp.bfloat16), dn_t,
                        preferred_element_type=jnp.float32)
    x += lax.dot_general(w1r_ref[...], gr_ref[...].astype(jnp.bfloat16), dn_t,
                         preferred_element_type=jnp.float32)
    el = el_ref[0]
    x += lax.dot_general(w1e_ref[...], el, (((0,), (0,)), ((), ())),
                         preferred_element_type=jnp.float32)
    x += b1_ref[...]
    hmid = (x * jax.nn.sigmoid(x)).astype(jnp.bfloat16)
    m = lax.dot_general(hmid, w2_ref[...], (((0,), (0,)), ((), ())),
                        preferred_element_type=jnp.float32)
    o_ref[...] = m + b2_ref[...]


def _mlp(gathered, el3, w1s, w1r, w1e, b1c, w2, b2r):
    return pl.pallas_call(
        _mlp_body,
        grid=(NB,),
        in_specs=[
            pl.BlockSpec((EB, HIDDEN), lambda i: (i, 0)),
            pl.BlockSpec((EB, HIDDEN), lambda i: (i + NB, 0)),
            pl.BlockSpec((1, 1, EB), lambda i: (i, 0, 0)),
            pl.BlockSpec((HIDDEN, HIDDEN), lambda i: (0, 0)),
            pl.BlockSpec((HIDDEN, HIDDEN), lambda i: (0, 0)),
            pl.BlockSpec((1, HIDDEN), lambda i: (0, 0)),
            pl.BlockSpec((HIDDEN, 1), lambda i: (0, 0)),
            pl.BlockSpec((HIDDEN, HIDDEN), lambda i: (0, 0)),
            pl.BlockSpec((1, HIDDEN), lambda i: (0, 0)),
        ],
        out_specs=pl.BlockSpec((EB, HIDDEN), lambda i: (i, 0)),
        out_shape=jax.ShapeDtypeStruct((E_PAD, HIDDEN), jnp.float32),
    )(gathered, gathered, el3, w1s, w1r, w1e, b1c, w2, b2r)


def _final_body(h_ref, p_ref, o_ref):
    o_ref[:, :HIDDEN] = h_ref[:, :HIDDEN] + p_ref[0] + p_ref[1]
    o_ref[:, HIDDEN:] = h_ref[:, HIDDEN:]


def _finalize(h, part):
    n, f = h.shape
    rb = 1000
    return pl.pallas_call(
        _final_body,
        grid=(n // rb,),
        in_specs=[
            pl.BlockSpec((rb, f), lambda i: (i, 0)),
            pl.BlockSpec((2, rb, HIDDEN), lambda i: (0, i, 0)),
        ],
        out_specs=pl.BlockSpec((rb, f), lambda i: (i, 0)),
        out_shape=jax.ShapeDtypeStruct((n, f), jnp.float32),
    )(h, part)


def kernel(h, edge_index, edge_len, W1, b1, W2, b2):
    scalars = h[:, :HIDDEN]
    sender = edge_index[0].astype(jnp.int32)
    receiver = edge_index[1].astype(jnp.int32)
    e = sender.shape[0]
    pad = E_PAD - e
    sender_p = jnp.pad(sender, (0, pad))
    receiver_p = jnp.pad(receiver, (0, pad), constant_values=N_NODES)
    el_p = jnp.pad(edge_len.astype(jnp.float32), (0, pad))
    idx_all = jnp.concatenate([sender_p, receiver_p]).reshape(G // W, W)

    gathered = _gather(scalars, idx_all)
    msgs = _mlp(
        gathered,
        el_p.reshape(NB, 1, EB),
        W1[:HIDDEN].astype(jnp.bfloat16),
        W1[HIDDEN:2 * HIDDEN].astype(jnp.bfloat16),
        W1[2 * HIDDEN:],
        b1.reshape(HIDDEN, 1),
        W2.astype(jnp.bfloat16),
        b2.reshape(1, HIDDEN),
    )
    part = _scatter_add(
        msgs,
        receiver_p.reshape(1, E_PAD),
        jnp.zeros((ACC_ROWS, HIDDEN), jnp.float32),
    )
    return _finalize(h, part)


# speedup vs baseline: 3.0321x; 1.0458x over previous
import functools

import jax
import jax.numpy as jnp
from jax import lax
from jax.experimental import pallas as pl
from jax.experimental.pallas import tpu as pltpu
from jax.experimental.pallas import tpu_sc as plsc

HIDDEN = 128
N_NODES = 10000
E_PAD = 327680
ACC_ROWS = 10240
EB = 2048
W = 128
N_SUBCORES = 16
NW = 2 * N_SUBCORES
NBUF = 4

C = 4
EC = E_PAD // C
GC = 2 * EC
CW = GC // W
WSTEPS = CW // NW
NBC = EC // EB
SW = E_PAD // W
SWC = SW // C
SWW = SWC // NW


def _sc_mesh():
    return plsc.VectorSubcoreMesh(core_axis_name="c", subcore_axis_name="s")


def _gather(table, idx):
    @functools.partial(
        pl.kernel,
        out_type=jax.ShapeDtypeStruct((GC, HIDDEN), jnp.float32),
        mesh=_sc_mesh(),
        scratch_types=[
            pltpu.VMEM((WSTEPS, W), jnp.int32),
            pltpu.VMEM((NBUF, W, HIDDEN), jnp.float32),
            pltpu.SemaphoreType.DMA((NBUF,)),
            pltpu.SemaphoreType.DMA((NBUF,)),
        ],
    )
    def kern(table_hbm, idx_hbm, out_hbm, idx_v, bufs, gsem, osem):
        wid = lax.axis_index("c") * N_SUBCORES + lax.axis_index("s")
        pltpu.sync_copy(idx_hbm.at[pl.ds(wid * WSTEPS, WSTEPS)], idx_v)

        def out_slot(w):
            return out_hbm.at[pl.ds((wid * WSTEPS + w) * W, W)]

        for b in range(NBUF):
            pltpu.async_copy(table_hbm.at[idx_v.at[b]], bufs.at[b], gsem.at[b])

        @pl.loop(0, WSTEPS // NBUF)
        def _(k):
            for b in range(NBUF):
                w = k * NBUF + b
                pltpu.make_async_copy(
                    table_hbm.at[idx_v.at[w]], bufs.at[b], gsem.at[b]).wait()
                pltpu.async_copy(bufs.at[b], out_slot(w), osem.at[b])

                @pl.when(k < WSTEPS // NBUF - 1)
                def _():
                    pltpu.make_async_copy(
                        bufs.at[b], out_slot(w), osem.at[b]).wait()
                    pltpu.async_copy(table_hbm.at[idx_v.at[w + NBUF]],
                                     bufs.at[b], gsem.at[b])

        for b in range(NBUF):
            pltpu.make_async_copy(
                bufs.at[b], out_slot(WSTEPS - NBUF + b), osem.at[b]).wait()

    return kern(table, idx)


def _scatter_add(msgs_chunks, ridx, zeros):
    @functools.partial(
        pl.kernel,
        out_type=jax.ShapeDtypeStruct((2, ACC_ROWS, HIDDEN), jnp.float32),
        mesh=_sc_mesh(),
        scratch_types=[
            pltpu.VMEM_SHARED((ACC_ROWS, HIDDEN), jnp.float32),
            pltpu.VMEM((SWW, W), jnp.int32),
            pltpu.VMEM((2, W, HIDDEN), jnp.float32),
            pltpu.SemaphoreType.DMA((2,)),
        ],
    )
    def kern(m0, m1, m2, m3, ridx_hbm, zeros_hbm, part_hbm,
             acc, idx_v, mbuf, msem):
        c = lax.axis_index("c")
        s = lax.axis_index("s")
        wid = c * N_SUBCORES + s
        stripe = ACC_ROWS // N_SUBCORES
        r0 = s * stripe
        pltpu.sync_copy(zeros_hbm.at[pl.ds(r0, stripe)],
                        acc.at[pl.ds(r0, stripe)])
        plsc.subcore_barrier()

        for ci, m in enumerate((m0, m1, m2, m3)):
            pltpu.sync_copy(ridx_hbm.at[ci * NW + wid], idx_v)

            def mslot(t):
                return m.at[pl.ds((wid * SWW + t) * W, W)]

            for b in range(2):
                pltpu.async_copy(mslot(b), mbuf.at[b], msem.at[b])
            for t in range(SWW):
                b = t % 2
                pltpu.make_async_copy(mslot(t), mbuf.at[b], msem.at[b]).wait()
                pltpu.sync_copy(mbuf.at[b], acc.at[idx_v.at[t]], add=True)
                if t + 2 < SWW:
                    pltpu.async_copy(mslot(t + 2), mbuf.at[b], msem.at[b])

        plsc.subcore_barrier()
        pltpu.sync_copy(acc.at[pl.ds(r0, stripe)],
                        part_hbm.at[c, pl.ds(r0, stripe)])

    return kern(*msgs_chunks, ridx, zeros)


def _mlp_body(gs_ref, gr_ref, el_ref, w1s_ref, w1r_ref, w1e_ref, b1_ref,
              w2_ref, b2_ref, o_ref):
    dn_t = (((0,), (1,)), ((), ()))
    x = lax.dot_general(w1s_ref[...], gs_ref[...].astype(jnp.bfloat16), dn_t,
                        preferred_element_type=jnp.float32)
    x += lax.dot_general(w1r_ref[...], gr_ref[...].astype(jnp.bfloat16), dn_t,
                         preferred_element_type=jnp.float32)
    el = el_ref[0]
    x += lax.dot_general(w1e_ref[...], el, (((0,), (0,)), ((), ())),
                         preferred_element_type=jnp.float32)
    x += b1_ref[...]
    hmid = (x * jax.nn.sigmoid(x)).astype(jnp.bfloat16)
    m = lax.dot_general(hmid, w2_ref[...], (((0,), (0,)), ((), ())),
                        preferred_element_type=jnp.float32)
    o_ref[...] = m + b2_ref[...]


def _mlp(gathered, el3, w1s, w1r, w1e, b1c, w2, b2r):
    return pl.pallas_call(
        _mlp_body,
        grid=(NBC,),
        in_specs=[
            pl.BlockSpec((EB, HIDDEN), lambda i: (i, 0)),
            pl.BlockSpec((EB, HIDDEN), lambda i: (i + NBC, 0)),
            pl.BlockSpec((1, 1, EB), lambda i: (i, 0, 0)),
            pl.BlockSpec((HIDDEN, HIDDEN), lambda i: (0, 0)),
            pl.BlockSpec((HIDDEN, HIDDEN), lambda i: (0, 0)),
            pl.BlockSpec((1, HIDDEN), lambda i: (0, 0)),
            pl.BlockSpec((HIDDEN, 1), lambda i: (0, 0)),
            pl.BlockSpec((HIDDEN, HIDDEN), lambda i: (0, 0)),
            pl.BlockSpec((1, HIDDEN), lambda i: (0, 0)),
        ],
        out_specs=pl.BlockSpec((EB, HIDDEN), lambda i: (i, 0)),
        out_shape=jax.ShapeDtypeStruct((EC, HIDDEN), jnp.float32),
    )(gathered, gathered, el3, w1s, w1r, w1e, b1c, w2, b2r)


def _final_body(h_ref, p_ref, o_ref):
    o_ref[:, :HIDDEN] = h_ref[:, :HIDDEN] + p_ref[0] + p_ref[1]
    o_ref[:, HIDDEN:] = h_ref[:, HIDDEN:]


def _finalize(h, part):
    n, f = h.shape
    rb = 1000
    return pl.pallas_call(
        _final_body,
        grid=(n // rb,),
        in_specs=[
            pl.BlockSpec((rb, f), lambda i: (i, 0)),
            pl.BlockSpec((2, rb, HIDDEN), lambda i: (0, i, 0)),
        ],
        out_specs=pl.BlockSpec((rb, f), lambda i: (i, 0)),
        out_shape=jax.ShapeDtypeStruct((n, f), jnp.float32),
    )(h, part)


def kernel(h, edge_index, edge_len, W1, b1, W2, b2):
    scalars = h[:, :HIDDEN]
    sender = edge_index[0].astype(jnp.int32)
    receiver = edge_index[1].astype(jnp.int32)
    e = sender.shape[0]
    pad = E_PAD - e
    sender_p = jnp.pad(sender, (0, pad))
    receiver_p = jnp.pad(receiver, (0, pad), constant_values=N_NODES)
    el_p = jnp.pad(edge_len.astype(jnp.float32), (0, pad))

    w1s = W1[:HIDDEN].astype(jnp.bfloat16)
    w1r = W1[HIDDEN:2 * HIDDEN].astype(jnp.bfloat16)
    w1e = W1[2 * HIDDEN:]
    b1c = b1.reshape(HIDDEN, 1)
    w2 = W2.astype(jnp.bfloat16)
    b2r = b2.reshape(1, HIDDEN)

    msgs_chunks = []
    for ci in range(C):
        sl = slice(ci * EC, (ci + 1) * EC)
        idx_c = jnp.concatenate([sender_p[sl], receiver_p[sl]]).reshape(CW, W)
        gathered = _gather(scalars, idx_c)
        msgs_chunks.append(
            _mlp(gathered, el_p[sl].reshape(NBC, 1, EB),
                 w1s, w1r, w1e, b1c, w2, b2r))

    part = _scatter_add(
        msgs_chunks,
        receiver_p.reshape(C * NW, SWW, W),
        jnp.zeros((ACC_ROWS, HIDDEN), jnp.float32),
    )
    return _finalize(h, part)


# speedup vs baseline: 5.7536x; 1.8975x over previous
import functools

import jax
import jax.numpy as jnp
from jax import lax
from jax.experimental import pallas as pl
from jax.experimental.pallas import tpu as pltpu
from jax.experimental.pallas import tpu_sc as plsc

HIDDEN = 128
N_NODES = 10000
E_PAD = 327680
ACC_ROWS = 10240
EB = 2048
W = 128
N_SUBCORES = 16
NW = 2 * N_SUBCORES
NBUF = 2

C = 4
EC = E_PAD // C
GC = 2 * EC
CW = GC // W
WSTEPS = CW // NW
NBC = EC // EB
SW = E_PAD // W
SWC = SW // C
SWW = SWC // NW


def _sc_mesh():
    return plsc.VectorSubcoreMesh(core_axis_name="c", subcore_axis_name="s")


def _gather(table, idx):
    @functools.partial(
        pl.kernel,
        out_type=jax.ShapeDtypeStruct((GC, HIDDEN), jnp.float32),
        mesh=_sc_mesh(),
        scratch_types=[
            pltpu.VMEM_SHARED((ACC_ROWS, HIDDEN), jnp.float32),
            pltpu.VMEM((WSTEPS, W), jnp.int32),
            pltpu.VMEM((NBUF, W, HIDDEN), jnp.float32),
            pltpu.SemaphoreType.DMA((NBUF,)),
            pltpu.SemaphoreType.DMA((NBUF,)),
        ],
    )
    def kern(table_hbm, idx_hbm, out_hbm, table_s, idx_v, bufs, gsem, osem):
        s = lax.axis_index("s")
        wid = lax.axis_index("c") * N_SUBCORES + s
        stripe = ACC_ROWS // N_SUBCORES
        r0 = s * stripe
        pltpu.sync_copy(table_hbm.at[pl.ds(r0, stripe)],
                        table_s.at[pl.ds(r0, stripe)])
        pltpu.sync_copy(idx_hbm.at[pl.ds(wid * WSTEPS, WSTEPS)], idx_v)
        plsc.subcore_barrier()

        def out_slot(w):
            return out_hbm.at[pl.ds((wid * WSTEPS + w) * W, W)]

        for b in range(NBUF):
            pltpu.async_copy(table_s.at[idx_v.at[b]], bufs.at[b], gsem.at[b])

        @pl.loop(0, WSTEPS // NBUF)
        def _(k):
            for b in range(NBUF):
                w = k * NBUF + b
                pltpu.make_async_copy(
                    table_s.at[idx_v.at[w]], bufs.at[b], gsem.at[b]).wait()
                pltpu.async_copy(bufs.at[b], out_slot(w), osem.at[b])

                @pl.when(k < WSTEPS // NBUF - 1)
                def _():
                    pltpu.make_async_copy(
                        bufs.at[b], out_slot(w), osem.at[b]).wait()
                    pltpu.async_copy(table_s.at[idx_v.at[w + NBUF]],
                                     bufs.at[b], gsem.at[b])

        for b in range(NBUF):
            pltpu.make_async_copy(
                bufs.at[b], out_slot(WSTEPS - NBUF + b), osem.at[b]).wait()

    return kern(table, idx)


def _scatter_add(msgs_chunks, ridx, zeros):
    @functools.partial(
        pl.kernel,
        out_type=jax.ShapeDtypeStruct((2, ACC_ROWS, HIDDEN), jnp.float32),
        mesh=_sc_mesh(),
        scratch_types=[
            pltpu.VMEM_SHARED((ACC_ROWS, HIDDEN), jnp.float32),
            pltpu.VMEM((SWW, W), jnp.int32),
            pltpu.VMEM((2, W, HIDDEN), jnp.float32),
            pltpu.SemaphoreType.DMA((2,)),
        ],
    )
    def kern(m0, m1, m2, m3, ridx_hbm, zeros_hbm, part_hbm,
             acc, idx_v, mbuf, msem):
        c = lax.axis_index("c")
        s = lax.axis_index("s")
        wid = c * N_SUBCORES + s
        stripe = ACC_ROWS // N_SUBCORES
        r0 = s * stripe
        pltpu.sync_copy(zeros_hbm.at[pl.ds(r0, stripe)],
                        acc.at[pl.ds(r0, stripe)])
        plsc.subcore_barrier()

        for ci, m in enumerate((m0, m1, m2, m3)):
            pltpu.sync_copy(ridx_hbm.at[ci * NW + wid], idx_v)

            def mslot(t):
                return m.at[pl.ds((wid * SWW + t) * W, W)]

            for b in range(2):
                pltpu.async_copy(mslot(b), mbuf.at[b], msem.at[b])
            for t in range(SWW):
                b = t % 2
                pltpu.make_async_copy(mslot(t), mbuf.at[b], msem.at[b]).wait()
                pltpu.sync_copy(mbuf.at[b], acc.at[idx_v.at[t]], add=True)
                if t + 2 < SWW:
                    pltpu.async_copy(mslot(t + 2), mbuf.at[b], msem.at[b])

        plsc.subcore_barrier()
        pltpu.sync_copy(acc.at[pl.ds(r0, stripe)],
                        part_hbm.at[c, pl.ds(r0, stripe)])

    return kern(*msgs_chunks, ridx, zeros)


def _mlp_body(gs_ref, gr_ref, el_ref, w1s_ref, w1r_ref, w1e_ref, b1_ref,
              w2_ref, b2_ref, o_ref):
    dn_t = (((0,), (1,)), ((), ()))
    x = lax.dot_general(w1s_ref[...], gs_ref[...].astype(jnp.bfloat16), dn_t,
                        preferred_element_type=jnp.float32)
    x += lax.dot_general(w1r_ref[...], gr_ref[...].astype(jnp.bfloat16), dn_t,
                         preferred_element_type=jnp.float32)
    el = el_ref[0]
    x += lax.dot_general(w1e_ref[...], el, (((0,), (0,)), ((), ())),
                         preferred_element_type=jnp.float32)
    x += b1_ref[...]
    hmid = (x * jax.nn.sigmoid(x)).astype(jnp.bfloat16)
    m = lax.dot_general(hmid, w2_ref[...], (((0,), (0,)), ((), ())),
                        preferred_element_type=jnp.float32)
    o_ref[...] = m + b2_ref[...]


def _mlp(gathered, el3, w1s, w1r, w1e, b1c, w2, b2r):
    return pl.pallas_call(
        _mlp_body,
        grid=(NBC,),
        in_specs=[
            pl.BlockSpec((EB, HIDDEN), lambda i: (i, 0)),
            pl.BlockSpec((EB, HIDDEN), lambda i: (i + NBC, 0)),
            pl.BlockSpec((1, 1, EB), lambda i: (i, 0, 0)),
            pl.BlockSpec((HIDDEN, HIDDEN), lambda i: (0, 0)),
            pl.BlockSpec((HIDDEN, HIDDEN), lambda i: (0, 0)),
            pl.BlockSpec((1, HIDDEN), lambda i: (0, 0)),
            pl.BlockSpec((HIDDEN, 1), lambda i: (0, 0)),
            pl.BlockSpec((HIDDEN, HIDDEN), lambda i: (0, 0)),
            pl.BlockSpec((1, HIDDEN), lambda i: (0, 0)),
        ],
        out_specs=pl.BlockSpec((EB, HIDDEN), lambda i: (i, 0)),
        out_shape=jax.ShapeDtypeStruct((EC, HIDDEN), jnp.float32),
    )(gathered, gathered, el3, w1s, w1r, w1e, b1c, w2, b2r)


def _final_body(h_ref, p_ref, o_ref):
    o_ref[:, :HIDDEN] = h_ref[:, :HIDDEN] + p_ref[0] + p_ref[1]
    o_ref[:, HIDDEN:] = h_ref[:, HIDDEN:]


def _finalize(h, part):
    n, f = h.shape
    rb = 1000
    return pl.pallas_call(
        _final_body,
        grid=(n // rb,),
        in_specs=[
            pl.BlockSpec((rb, f), lambda i: (i, 0)),
            pl.BlockSpec((2, rb, HIDDEN), lambda i: (0, i, 0)),
        ],
        out_specs=pl.BlockSpec((rb, f), lambda i: (i, 0)),
        out_shape=jax.ShapeDtypeStruct((n, f), jnp.float32),
    )(h, part)


def kernel(h, edge_index, edge_len, W1, b1, W2, b2):
    scalars = jnp.pad(h[:, :HIDDEN], ((0, ACC_ROWS - N_NODES), (0, 0)))
    sender = edge_index[0].astype(jnp.int32)
    receiver = edge_index[1].astype(jnp.int32)
    e = sender.shape[0]
    pad = E_PAD - e
    sender_p = jnp.pad(sender, (0, pad))
    receiver_p = jnp.pad(receiver, (0, pad), constant_values=N_NODES)
    el_p = jnp.pad(edge_len.astype(jnp.float32), (0, pad))

    w1s = W1[:HIDDEN].astype(jnp.bfloat16)
    w1r = W1[HIDDEN:2 * HIDDEN].astype(jnp.bfloat16)
    w1e = W1[2 * HIDDEN:]
    b1c = b1.reshape(HIDDEN, 1)
    w2 = W2.astype(jnp.bfloat16)
    b2r = b2.reshape(1, HIDDEN)

    msgs_chunks = []
    for ci in range(C):
        sl = slice(ci * EC, (ci + 1) * EC)
        idx_c = jnp.concatenate([sender_p[sl], receiver_p[sl]]).reshape(CW, W)
        gathered = _gather(scalars, idx_c)
        msgs_chunks.append(
            _mlp(gathered, el_p[sl].reshape(NBC, 1, EB),
                 w1s, w1r, w1e, b1c, w2, b2r))

    part = _scatter_add(
        msgs_chunks,
        receiver_p.reshape(C * NW, SWW, W),
        jnp.zeros((ACC_ROWS, HIDDEN), jnp.float32),
    )
    return _finalize(h, part)


# speedup vs baseline: 6.2065x; 1.0787x over previous
import functools

import jax
import jax.numpy as jnp
from jax import lax
from jax.experimental import pallas as pl
from jax.experimental.pallas import tpu as pltpu
from jax.experimental.pallas import tpu_sc as plsc

HIDDEN = 128
N_NODES = 10000
E_PAD = 327680
ACC_ROWS = 10240
EB = 2048
W = 128
N_SUBCORES = 16
NW = 2 * N_SUBCORES
NBUF = 2

C = 4
EC = E_PAD // C
GC = 2 * EC
CW = GC // W
WSTEPS = CW // NW
NBC = EC // EB
SW = E_PAD // W
SWC = SW // C
SWW = SWC // NW


def _sc_mesh():
    return plsc.VectorSubcoreMesh(core_axis_name="c", subcore_axis_name="s")


def _gather(table, idx):
    @functools.partial(
        pl.kernel,
        out_type=jax.ShapeDtypeStruct((GC, HIDDEN), jnp.float32),
        mesh=_sc_mesh(),
        scratch_types=[
            pltpu.VMEM_SHARED((ACC_ROWS, HIDDEN), jnp.float32),
            pltpu.VMEM((WSTEPS, W), jnp.int32),
            pltpu.VMEM((NBUF, W, HIDDEN), jnp.float32),
            pltpu.SemaphoreType.DMA((NBUF,)),
            pltpu.SemaphoreType.DMA((NBUF,)),
        ],
    )
    def kern(table_hbm, idx_hbm, out_hbm, table_s, idx_v, bufs, gsem, osem):
        s = lax.axis_index("s")
        wid = lax.axis_index("c") * N_SUBCORES + s
        stripe = ACC_ROWS // N_SUBCORES
        r0 = s * stripe
        pltpu.sync_copy(table_hbm.at[pl.ds(r0, stripe)],
                        table_s.at[pl.ds(r0, stripe)])
        pltpu.sync_copy(idx_hbm.at[pl.ds(wid * WSTEPS, WSTEPS)], idx_v)
        plsc.subcore_barrier()

        def out_slot(w):
            return out_hbm.at[pl.ds((wid * WSTEPS + w) * W, W)]

        for b in range(NBUF):
            pltpu.async_copy(table_s.at[idx_v.at[b]], bufs.at[b], gsem.at[b])

        @pl.loop(0, WSTEPS // NBUF)
        def _(k):
            for b in range(NBUF):
                w = k * NBUF + b
                pltpu.make_async_copy(
                    table_s.at[idx_v.at[w]], bufs.at[b], gsem.at[b]).wait()
                pltpu.async_copy(bufs.at[b], out_slot(w), osem.at[b])

                @pl.when(k < WSTEPS // NBUF - 1)
                def _():
                    pltpu.make_async_copy(
                        bufs.at[b], out_slot(w), osem.at[b]).wait()
                    pltpu.async_copy(table_s.at[idx_v.at[w + NBUF]],
                                     bufs.at[b], gsem.at[b])

        for b in range(NBUF):
            pltpu.make_async_copy(
                bufs.at[b], out_slot(WSTEPS - NBUF + b), osem.at[b]).wait()

    return kern(table, idx)


def _scatter_add(msgs, ridx, zeros):
    @functools.partial(
        pl.kernel,
        out_type=jax.ShapeDtypeStruct((2, ACC_ROWS, HIDDEN), jnp.float32),
        mesh=_sc_mesh(),
        scratch_types=[
            pltpu.VMEM_SHARED((ACC_ROWS, HIDDEN), jnp.float32),
            pltpu.VMEM((SWW, W), jnp.int32),
            pltpu.VMEM((2, W, HIDDEN), jnp.float32),
            pltpu.SemaphoreType.DMA((2,)),
        ],
    )
    def kern(m, ridx_hbm, zeros_hbm, part_hbm, acc, idx_v, mbuf, msem):
        c = lax.axis_index("c")
        s = lax.axis_index("s")
        wid = c * N_SUBCORES + s
        stripe = ACC_ROWS // N_SUBCORES
        r0 = s * stripe
        pltpu.sync_copy(zeros_hbm.at[pl.ds(r0, stripe)],
                        acc.at[pl.ds(r0, stripe)])
        pltpu.sync_copy(ridx_hbm.at[wid], idx_v)
        plsc.subcore_barrier()

        def mslot(t):
            return m.at[pl.ds((wid * SWW + t) * W, W)]

        for b in range(2):
            pltpu.async_copy(mslot(b), mbuf.at[b], msem.at[b])
        for t in range(SWW):
            b = t % 2
            pltpu.make_async_copy(mslot(t), mbuf.at[b], msem.at[b]).wait()
            pltpu.sync_copy(mbuf.at[b], acc.at[idx_v.at[t]], add=True)
            if t + 2 < SWW:
                pltpu.async_copy(mslot(t + 2), mbuf.at[b], msem.at[b])

        plsc.subcore_barrier()
        pltpu.sync_copy(acc.at[pl.ds(r0, stripe)],
                        part_hbm.at[c, pl.ds(r0, stripe)])

    return kern(msgs, ridx, zeros)


def _mlp_body(gs_ref, gr_ref, el_ref, w1s_ref, w1r_ref, w1e_ref, b1_ref,
              w2_ref, b2_ref, o_ref):
    dn_t = (((0,), (1,)), ((), ()))
    x = lax.dot_general(w1s_ref[...], gs_ref[...].astype(jnp.bfloat16), dn_t,
                        preferred_element_type=jnp.float32)
    x += lax.dot_general(w1r_ref[...], gr_ref[...].astype(jnp.bfloat16), dn_t,
                         preferred_element_type=jnp.float32)
    el = el_ref[0]
    x += lax.dot_general(w1e_ref[...], el, (((0,), (0,)), ((), ())),
                         preferred_element_type=jnp.float32)
    x += b1_ref[...]
    hmid = (x * jax.nn.sigmoid(x)).astype(jnp.bfloat16)
    m = lax.dot_general(hmid, w2_ref[...], (((0,), (0,)), ((), ())),
                        preferred_element_type=jnp.float32)
    o_ref[...] = m + b2_ref[...]


def _mlp(gathered, el3, w1s, w1r, w1e, b1c, w2, b2r):
    return pl.pallas_call(
        _mlp_body,
        grid=(NBC,),
        in_specs=[
            pl.BlockSpec((EB, HIDDEN), lambda i: (i, 0)),
            pl.BlockSpec((EB, HIDDEN), lambda i: (i + NBC, 0)),
            pl.BlockSpec((1, 1, EB), lambda i: (i, 0, 0)),
            pl.BlockSpec((HIDDEN, HIDDEN), lambda i: (0, 0)),
            pl.BlockSpec((HIDDEN, HIDDEN), lambda i: (0, 0)),
            pl.BlockSpec((1, HIDDEN), lambda i: (0, 0)),
            pl.BlockSpec((HIDDEN, 1), lambda i: (0, 0)),
            pl.BlockSpec((HIDDEN, HIDDEN), lambda i: (0, 0)),
            pl.BlockSpec((1, HIDDEN), lambda i: (0, 0)),
        ],
        out_specs=pl.BlockSpec((EB, HIDDEN), lambda i: (i, 0)),
        out_shape=jax.ShapeDtypeStruct((EC, HIDDEN), jnp.float32),
    )(gathered, gathered, el3, w1s, w1r, w1e, b1c, w2, b2r)


def _final_body(h_ref, p0_ref, p1_ref, p2_ref, p3_ref, o_ref):
    agg = p0_ref[0] + p0_ref[1] + p1_ref[0] + p1_ref[1]
    agg += p2_ref[0] + p2_ref[1] + p3_ref[0] + p3_ref[1]
    o_ref[:, :HIDDEN] = h_ref[:, :HIDDEN] + agg
    o_ref[:, HIDDEN:] = h_ref[:, HIDDEN:]


def _finalize(h, parts):
    n, f = h.shape
    rb = 1000
    pspec = pl.BlockSpec((2, rb, HIDDEN), lambda i: (0, i, 0))
    return pl.pallas_call(
        _final_body,
        grid=(n // rb,),
        in_specs=[pl.BlockSpec((rb, f), lambda i: (i, 0))] + [pspec] * C,
        out_specs=pl.BlockSpec((rb, f), lambda i: (i, 0)),
        out_shape=jax.ShapeDtypeStruct((n, f), jnp.float32),
    )(h, *parts)


def kernel(h, edge_index, edge_len, W1, b1, W2, b2):
    scalars = jnp.pad(h[:, :HIDDEN], ((0, ACC_ROWS - N_NODES), (0, 0)))
    sender = edge_index[0].astype(jnp.int32)
    receiver = edge_index[1].astype(jnp.int32)
    e = sender.shape[0]
    pad = E_PAD - e
    sender_p = jnp.pad(sender, (0, pad))
    receiver_p = jnp.pad(receiver, (0, pad), constant_values=N_NODES)
    el_p = jnp.pad(edge_len.astype(jnp.float32), (0, pad))

    w1s = W1[:HIDDEN].astype(jnp.bfloat16)
    w1r = W1[HIDDEN:2 * HIDDEN].astype(jnp.bfloat16)
    w1e = W1[2 * HIDDEN:]
    b1c = b1.reshape(HIDDEN, 1)
    w2 = W2.astype(jnp.bfloat16)
    b2r = b2.reshape(1, HIDDEN)

    zeros = jnp.zeros((ACC_ROWS, HIDDEN), jnp.float32)
    parts = []
    for ci in range(C):
        sl = slice(ci * EC, (ci + 1) * EC)
        idx_c = jnp.concatenate([sender_p[sl], receiver_p[sl]]).reshape(CW, W)
        gathered = _gather(scalars, idx_c)
        msgs = _mlp(gathered, el_p[sl].reshape(NBC, 1, EB),
                    w1s, w1r, w1e, b1c, w2, b2r)
        parts.append(
            _scatter_add(msgs, receiver_p[sl].reshape(NW, SWW, W), zeros))
    return _finalize(h, parts)
